# Initial kernel scaffold; baseline (speedup 1.0000x reference)
import jax
import jax.numpy as jnp
from jax.experimental import pallas as pl


def kernel(x, edge_index_sc, edge_weight_sc, edge_index_fc, edge_weight_fc, batch, params):
    raise NotImplementedError("write your pallas kernel here")


# speedup vs baseline: 7.3760x; 7.3760x over previous
import functools

import jax
import jax.numpy as jnp
from jax import lax
from jax.experimental import pallas as pl
from jax.experimental.pallas import tpu as pltpu
from jax.experimental.pallas import tpu_sc as plsc

N = 10000
E = 320000
D = 128
H = 128
B = 64
OUT = 2
EPS = 1e-5

NT = 16
NP = 10240
RPT = NP // NT
C = 128
EPT_CH = 157
EPT = C * EPT_CH
EP = NT * EPT

def _sc_deg_body(dst_hbm, w_hbm, deg_hbm, zbuf, didx, wv, deg_sh):
    c = lax.axis_index("c")
    t = lax.axis_index("s")

    for i in range(RPT // 16):
        zbuf[pl.ds(i * 16, 16)] = jnp.zeros((16,), jnp.float32)
    pltpu.sync_copy(zbuf, deg_sh.at[pl.ds(t * RPT, RPT)])
    plsc.subcore_barrier()

    def chunk_body(k, _):
        base = c * EP + t * EPT + k * C
        pltpu.sync_copy(dst_hbm.at[pl.ds(base, C)], didx)
        pltpu.sync_copy(w_hbm.at[pl.ds(base, C)], wv)
        pltpu.sync_copy(wv, deg_sh.at[didx], add=True)
        return 0

    lax.fori_loop(0, EPT_CH, chunk_body, 0)
    plsc.subcore_barrier()
    pltpu.sync_copy(deg_sh.at[pl.ds(t * RPT, RPT)],
                    deg_hbm.at[c, pl.ds(t * RPT, RPT)])


def _sc_mp_body(hw_hbm, src_hbm, dst_hbm, w_hbm, acc_hbm,
                rows, sidx, didx, wv, acc_sh, sem):
    c = lax.axis_index("c")
    t = lax.axis_index("s")

    def zrow(i, _):
        for f in range(H // 16):
            rows[i, pl.ds(f * 16, 16)] = jnp.zeros((16,), jnp.float32)
        return 0

    lax.fori_loop(0, C, zrow, 0)
    for j in range(RPT // C):
        pltpu.sync_copy(rows, acc_sh.at[pl.ds(t * RPT + j * C, C), :])
    plsc.subcore_barrier()

    def chunk_body(k, _):
        base = c * EP + t * EPT + k * C
        pltpu.sync_copy(src_hbm.at[pl.ds(base, C)], sidx)
        pltpu.sync_copy(dst_hbm.at[pl.ds(base, C)], didx)
        pltpu.sync_copy(w_hbm.at[pl.ds(base, C), :], wv)
        pltpu.async_copy(hw_hbm.at[sidx], rows, sem).wait()

        def scale(e, _):
            wb = wv[e, :]
            for f in range(H // 16):
                sl = pl.ds(f * 16, 16)
                rows[e, sl] = rows[e, sl] * wb
            return 0

        lax.fori_loop(0, C, scale, 0)
        pltpu.sync_copy(rows, acc_sh.at[didx], add=True)
        return 0

    lax.fori_loop(0, EPT_CH, chunk_body, 0)
    plsc.subcore_barrier()
    for j in range(RPT // C):
        r0 = t * RPT + j * C
        pltpu.sync_copy(acc_sh.at[pl.ds(r0, C), :],
                        acc_hbm.at[c, pl.ds(r0, C), :])


@functools.lru_cache(maxsize=None)
def _build_sc_kernels():
    mesh = plsc.VectorSubcoreMesh(core_axis_name="c", subcore_axis_name="s")
    deg = functools.partial(
        pl.kernel,
        out_type=jax.ShapeDtypeStruct((2, NP), jnp.float32),
        mesh=mesh,
        scratch_types=[
            pltpu.VMEM((RPT,), jnp.float32),
            pltpu.VMEM((C,), jnp.int32),
            pltpu.VMEM((C,), jnp.float32),
            pltpu.VMEM_SHARED((NP,), jnp.float32),
        ],
    )(_sc_deg_body)
    mp = functools.partial(
        pl.kernel,
        out_type=jax.ShapeDtypeStruct((2, NP, H), jnp.float32),
        mesh=mesh,
        scratch_types=[
            pltpu.VMEM((C, H), jnp.float32),
            pltpu.VMEM((C,), jnp.int32),
            pltpu.VMEM((C,), jnp.int32),
            pltpu.VMEM((C, 16), jnp.float32),
            pltpu.VMEM_SHARED((NP, H), jnp.float32),
            pltpu.SemaphoreType.DMA,
        ],
    )(_sc_mp_body)
    return deg, mp


def _sc_deg(dst_all, w_all):
    return _build_sc_kernels()[0](dst_all, w_all)


def _sc_mp(hw_flat, src_all, dst_all, w_all):
    return _build_sc_kernels()[1](hw_flat, src_all, dst_all, w_all)


_HI = lax.Precision.HIGHEST


def _dot(a, b):
    return jnp.dot(a, b, preferred_element_type=jnp.float32)


def _bn(h, g, b):
    m = jnp.mean(h, axis=0)
    v = jnp.mean((h - m) ** 2, axis=0)
    return g * (h - m) / jnp.sqrt(v + EPS) + b


def _tc_enc_body(x_ref, deg_ref, We1, be1, ge1, bbe1, We2, be2,
                 h_ref, dinv_ref):
    x = x_ref[:]
    h = _dot(x, We1[:]) + be1[:][None, :]
    h = _bn(h, ge1[:][None, :], bbe1[:][None, :])
    h = jnp.maximum(h, 0.0)
    h_ref[:] = _dot(h, We2[:]) + be2[:][None, :]
    for g in range(2):
        deg = deg_ref[g, pl.ds(0, N)] + 1.0
        dinv_ref[g] = jnp.where(deg > 0,
                                lax.rsqrt(jnp.maximum(deg, 1e-12)), 0.0)


_tc_enc = pl.pallas_call(
    _tc_enc_body,
    out_shape=[
        jax.ShapeDtypeStruct((N, H), jnp.float32),
        jax.ShapeDtypeStruct((2, N), jnp.float32),
    ],
)


def _tc_prep_body(h_ref, dinv_ref, W0, hw_ref):
    hw_ref[:] = _dot(h_ref[:], W0[:]) * dinv_ref[:][:, None]


_tc_prep = pl.pallas_call(
    _tc_prep_body,
    out_shape=jax.ShapeDtypeStruct((N, H), jnp.float32),
)


def _tc_mid_body(acc_ref, hw_ref, dinv_ref, b0, g0, b0bn, W1, hw1_ref):
    dinv = dinv_ref[:]
    acc = acc_ref[pl.ds(0, N), :]
    t = dinv[:, None] * (acc + hw_ref[:]) + b0[:][None, :]
    t = _bn(t, g0[:][None, :], b0bn[:][None, :])
    t = jnp.maximum(t, 0.0)
    hw1_ref[:] = _dot(t, W1[:]) * dinv[:, None]


_tc_mid = pl.pallas_call(
    _tc_mid_body,
    out_shape=jax.ShapeDtypeStruct((N, H), jnp.float32),
)


def _tc_lay2_body(acc_ref, hw_ref, dinv_ref, b1, g1, b1bn, h2_ref):
    dinv = dinv_ref[:]
    acc = acc_ref[pl.ds(0, N), :]
    t = dinv[:, None] * (acc + hw_ref[:]) + b1[:][None, :]
    t = _bn(t, g1[:][None, :], b1bn[:][None, :])
    h2_ref[:] = jnp.maximum(t, 0.0)


_tc_lay2 = pl.pallas_call(
    _tc_lay2_body,
    out_shape=jax.ShapeDtypeStruct((N, H), jnp.float32),
)


def _tc_head_body(h2s_ref, h2f_ref, batch_ref, Wg, bg,
                  Wp1, bp1, gp1, bp1bn, Wp2, bp2, gp2, bp2bn, Wp3, bp3,
                  out_ref):
    h2s, h2f = h2s_ref[:], h2f_ref[:]
    cat = jnp.concatenate([h2s, h2f], axis=1)
    gate = jax.nn.sigmoid(_dot(cat, Wg[:]) + bg[:][None, :])
    fused = gate * h2s + (1.0 - gate) * h2f
    onehot = (batch_ref[:] ==
              lax.broadcasted_iota(jnp.int32, (N, B), 1)).astype(jnp.float32)
    sums = lax.dot_general(onehot, fused, (((0,), (0,)), ((), ())),
                           preferred_element_type=jnp.float32, precision=_HI)
    cnts = jnp.sum(onehot, axis=0)
    gp = sums / jnp.maximum(cnts, 1.0)[:, None]
    z = _dot(gp, Wp1[:]) + bp1[:][None, :]
    z = _bn(z, gp1[:][None, :], bp1bn[:][None, :])
    z = jnp.maximum(z, 0.0)
    z = _dot(z, Wp2[:]) + bp2[:][None, :]
    z = _bn(z, gp2[:][None, :], bp2bn[:][None, :])
    z = jnp.maximum(z, 0.0)
    out_ref[:] = _dot(z, Wp3[:]) + bp3[:][None, :]


_tc_head = pl.pallas_call(
    _tc_head_body,
    out_shape=jax.ShapeDtypeStruct((B, OUT), jnp.float32),
)


def kernel(x, edge_index_sc, edge_weight_sc, edge_index_fc, edge_weight_fc,
           batch, params):
    p = params
    i32 = jnp.int32
    src_sc = edge_index_sc[0].astype(i32)
    dst_sc = edge_index_sc[1].astype(i32)
    src_fc = edge_index_fc[0].astype(i32)
    dst_fc = edge_index_fc[1].astype(i32)

    npad = EP - E
    pad_idx = (jnp.arange(npad, dtype=i32) * 61) % N
    zpad = jnp.zeros((npad,), jnp.float32)
    src_all = jnp.concatenate(
        [src_sc, pad_idx, src_fc + N, pad_idx + N])
    dst_all = jnp.concatenate([dst_sc, pad_idx, dst_fc, pad_idx])
    w_all = jnp.concatenate(
        [edge_weight_sc.astype(jnp.float32), zpad,
         edge_weight_fc.astype(jnp.float32), zpad])

    deg2 = _sc_deg(dst_all, w_all)
    w_exp = jnp.broadcast_to(w_all[:, None], (2 * EP, 16))

    h, dinv2 = _tc_enc(x, deg2, p['We1'], p['be1'], p['ge1'], p['bbe1'],
                       p['We2'], p['be2'])
    dinv = [dinv2[0], dinv2[1]]

    hw0 = [_tc_prep(h, dinv[0], p['Wsc0']), _tc_prep(h, dinv[1], p['Wfc0'])]
    acc0 = _sc_mp(jnp.concatenate(hw0, axis=0), src_all, dst_all, w_exp)

    hw1 = [_tc_mid(acc0[0], hw0[0], dinv[0], p['bsc0'], p['g_sc0'],
                   p['b_sc0'], p['Wsc1']),
           _tc_mid(acc0[1], hw0[1], dinv[1], p['bfc0'], p['g_fc0'],
                   p['b_fc0'], p['Wfc1'])]
    acc1 = _sc_mp(jnp.concatenate(hw1, axis=0), src_all, dst_all, w_exp)

    h2s = _tc_lay2(acc1[0], hw1[0], dinv[0], p['bsc1'], p['g_sc1'], p['b_sc1'])
    h2f = _tc_lay2(acc1[1], hw1[1], dinv[1], p['bfc1'], p['g_fc1'], p['b_fc1'])

    return _tc_head(h2s, h2f, batch.astype(i32).reshape(N, 1),
                    p['Wg'], p['bg'],
                    p['Wp1'], p['bp1'], p['gp1'], p['bp1_bn'],
                    p['Wp2'], p['bp2'], p['gp2'], p['bp2_bn'],
                    p['Wp3'], p['bp3'])


# speedup vs baseline: 15.9821x; 2.1668x over previous
import functools

import jax
import jax.numpy as jnp
from jax import lax
from jax.experimental import pallas as pl
from jax.experimental.pallas import tpu as pltpu
from jax.experimental.pallas import tpu_sc as plsc

N = 10000
E = 320000
D = 128
H = 128
B = 64
OUT = 2
EPS = 1e-5

NT = 16
NP = 10240
RPT = NP // NT
C = 128
EPT_CH = 157
EPT = C * EPT_CH
EP = NT * EPT

def _sc_deg_body(dst_hbm, w_hbm, deg_hbm, zbuf, didx, wv, deg_sh):
    c = lax.axis_index("c")
    t = lax.axis_index("s")

    for i in range(RPT // 16):
        zbuf[pl.ds(i * 16, 16)] = jnp.zeros((16,), jnp.float32)
    pltpu.sync_copy(zbuf, deg_sh.at[pl.ds(t * RPT, RPT)])
    plsc.subcore_barrier()

    def chunk_body(k, _):
        pltpu.sync_copy(dst_hbm.at[c, t, k], didx.at[0])
        pltpu.sync_copy(w_hbm.at[c, t, k], wv.at[0])
        pltpu.sync_copy(wv.at[0], deg_sh.at[didx.at[0]], add=True)
        return 0

    lax.fori_loop(0, EPT_CH, chunk_body, 0)
    plsc.subcore_barrier()
    pltpu.sync_copy(deg_sh.at[pl.ds(t * RPT, RPT)],
                    deg_hbm.at[c, pl.ds(t * RPT, RPT)])


def _sc_mp_body(hw_hbm, src_hbm, dst_hbm, w_hbm, acc_hbm,
                r0, r1, sidx, didx, wv, acc_sh,
                sg0, sg1, ss0, ss1, si0, si1, si2):
    c = lax.axis_index("c")
    t = lax.axis_index("s")
    rows = (r0, r1)
    sgs = (sg0, sg1)
    sss = (ss0, ss1)
    sis = (si0, si1, si2)

    def zrow(i, _):
        for f in range(H // 16):
            r0[i, pl.ds(f * 16, 16)] = jnp.zeros((16,), jnp.float32)
        return 0

    lax.fori_loop(0, C, zrow, 0)
    for j in range(RPT // C):
        pltpu.sync_copy(r0, acc_sh.at[pl.ds(t * RPT + j * C, C), :])
    plsc.subcore_barrier()

    def start_idx(k, ib):
        pltpu.make_async_copy(src_hbm.at[c, t, k], sidx.at[ib], sis[ib]).start()
        pltpu.make_async_copy(dst_hbm.at[c, t, k], didx.at[ib], sis[ib]).start()
        pltpu.make_async_copy(w_hbm.at[c, t, k], wv.at[ib], sis[ib]).start()

    def wait_idx(k, ib):
        pltpu.make_async_copy(src_hbm.at[c, t, k], sidx.at[ib], sis[ib]).wait()
        pltpu.make_async_copy(dst_hbm.at[c, t, k], didx.at[ib], sis[ib]).wait()
        pltpu.make_async_copy(w_hbm.at[c, t, k], wv.at[ib], sis[ib]).wait()

    def start_gather(rb, ib):
        pltpu.make_async_copy(hw_hbm.at[sidx.at[ib]], rows[rb], sgs[rb]).start()

    def wait_gather(rb, ib):
        pltpu.make_async_copy(hw_hbm.at[sidx.at[ib]], rows[rb], sgs[rb]).wait()

    def start_scatter(rb, ib):
        pltpu.async_copy(rows[rb], acc_sh.at[didx.at[ib]], sss[rb], add=True)

    def wait_scatter(rb, ib):
        pltpu.make_async_copy(rows[rb], acc_sh.at[didx.at[ib]], sss[rb]).wait()

    def scale(rb, ib):
        rbuf = rows[rb]

        def body(q, _):
            w16 = wv[ib, pl.ds(q * 16, 16)]
            for j in range(16):
                wb = jnp.full((16,), w16[j], jnp.float32)
                e = q * 16 + j
                for f in range(H // 16):
                    sl = pl.ds(f * 16, 16)
                    rbuf[e, sl] = rbuf[e, sl] * wb
            return 0

        lax.fori_loop(0, C // 16, body, 0)

    LAST = EPT_CH - 1
    start_idx(0, 0)
    start_idx(1, 1)
    wait_idx(0, 0)
    start_gather(0, 0)

    def step(kk, rb, ib):
        rb1 = (rb + 1) % 2
        ib1 = (ib + 1) % 3
        ib2 = (ib + 2) % 3
        wait_gather(rb, ib)
        scale(rb, ib)
        start_scatter(rb, ib)

        @pl.when(kk + 1 <= LAST)
        def _():
            @pl.when(kk >= 1)
            def _():
                wait_scatter(rb1, ib2)

            wait_idx(kk + 1, ib1)
            start_gather(rb1, ib1)

        @pl.when(kk + 2 <= LAST)
        def _():
            start_idx(kk + 2, ib2)

    def loop_body(k6, _):
        k0 = 6 * k6
        for i in range(6):
            step(k0 + i, i % 2, i % 3)
        return 0

    lax.fori_loop(0, LAST // 6, loop_body, 0)
    wait_gather(0, 0)
    scale(0, 0)
    start_scatter(0, 0)
    wait_scatter(1, 2)
    wait_scatter(0, 0)
    plsc.subcore_barrier()
    for j in range(RPT // C):
        rr = t * RPT + j * C
        pltpu.sync_copy(acc_sh.at[pl.ds(rr, C), :],
                        acc_hbm.at[c, pl.ds(rr, C), :])


@functools.lru_cache(maxsize=None)
def _build_sc_kernels():
    mesh = plsc.VectorSubcoreMesh(core_axis_name="c", subcore_axis_name="s")
    deg = functools.partial(
        pl.kernel,
        out_type=jax.ShapeDtypeStruct((2, NP), jnp.float32),
        mesh=mesh,
        scratch_types=[
            pltpu.VMEM((RPT,), jnp.float32),
            pltpu.VMEM((1, C), jnp.int32),
            pltpu.VMEM((1, C), jnp.float32),
            pltpu.VMEM_SHARED((NP,), jnp.float32),
        ],
    )(_sc_deg_body)
    mp = functools.partial(
        pl.kernel,
        out_type=jax.ShapeDtypeStruct((2, NP, H), jnp.float32),
        mesh=mesh,
        scratch_types=[
            pltpu.VMEM((C, H), jnp.float32),
            pltpu.VMEM((C, H), jnp.float32),
            pltpu.VMEM((3, C), jnp.int32),
            pltpu.VMEM((3, C), jnp.int32),
            pltpu.VMEM((3, C), jnp.float32),
            pltpu.VMEM_SHARED((NP, H), jnp.float32),
            pltpu.SemaphoreType.DMA,
            pltpu.SemaphoreType.DMA,
            pltpu.SemaphoreType.DMA,
            pltpu.SemaphoreType.DMA,
            pltpu.SemaphoreType.DMA,
            pltpu.SemaphoreType.DMA,
            pltpu.SemaphoreType.DMA,
        ],
    )(_sc_mp_body)
    return deg, mp


def _sc_deg(dst_r, w_r):
    return _build_sc_kernels()[0](dst_r, w_r)


def _sc_mp(hw_flat, src_r, dst_r, w_r):
    return _build_sc_kernels()[1](hw_flat, src_r, dst_r, w_r)


_HI = lax.Precision.HIGHEST


def _dot(a, b):
    return jnp.dot(a, b, preferred_element_type=jnp.float32)


def _bn(h, g, b):
    m = jnp.mean(h, axis=0)
    v = jnp.mean((h - m) ** 2, axis=0)
    return g * (h - m) / jnp.sqrt(v + EPS) + b


def _tc_enc_body(x_ref, deg_ref, We1, be1, ge1, bbe1, We2, be2,
                 h_ref, dinv_ref):
    x = x_ref[:]
    h = _dot(x, We1[:]) + be1[:][None, :]
    h = _bn(h, ge1[:][None, :], bbe1[:][None, :])
    h = jnp.maximum(h, 0.0)
    h_ref[:] = _dot(h, We2[:]) + be2[:][None, :]
    for g in range(2):
        deg = deg_ref[g, pl.ds(0, N)] + 1.0
        dinv_ref[g] = jnp.where(deg > 0,
                                lax.rsqrt(jnp.maximum(deg, 1e-12)), 0.0)


_tc_enc = pl.pallas_call(
    _tc_enc_body,
    out_shape=[
        jax.ShapeDtypeStruct((N, H), jnp.float32),
        jax.ShapeDtypeStruct((2, N), jnp.float32),
    ],
)


def _tc_prep_body(h_ref, dinv_ref, W0, hw_ref):
    hw_ref[:] = _dot(h_ref[:], W0[:]) * dinv_ref[:][:, None]


_tc_prep = pl.pallas_call(
    _tc_prep_body,
    out_shape=jax.ShapeDtypeStruct((N, H), jnp.float32),
)


def _tc_mid_body(acc_ref, hw_ref, dinv_ref, b0, g0, b0bn, W1, hw1_ref):
    dinv = dinv_ref[:]
    acc = acc_ref[pl.ds(0, N), :]
    t = dinv[:, None] * (acc + hw_ref[:]) + b0[:][None, :]
    t = _bn(t, g0[:][None, :], b0bn[:][None, :])
    t = jnp.maximum(t, 0.0)
    hw1_ref[:] = _dot(t, W1[:]) * dinv[:, None]


_tc_mid = pl.pallas_call(
    _tc_mid_body,
    out_shape=jax.ShapeDtypeStruct((N, H), jnp.float32),
)


def _tc_lay2_body(acc_ref, hw_ref, dinv_ref, b1, g1, b1bn, h2_ref):
    dinv = dinv_ref[:]
    acc = acc_ref[pl.ds(0, N), :]
    t = dinv[:, None] * (acc + hw_ref[:]) + b1[:][None, :]
    t = _bn(t, g1[:][None, :], b1bn[:][None, :])
    h2_ref[:] = jnp.maximum(t, 0.0)


_tc_lay2 = pl.pallas_call(
    _tc_lay2_body,
    out_shape=jax.ShapeDtypeStruct((N, H), jnp.float32),
)


def _tc_head_body(h2s_ref, h2f_ref, batch_ref, Wg, bg,
                  Wp1, bp1, gp1, bp1bn, Wp2, bp2, gp2, bp2bn, Wp3, bp3,
                  out_ref):
    h2s, h2f = h2s_ref[:], h2f_ref[:]
    cat = jnp.concatenate([h2s, h2f], axis=1)
    gate = jax.nn.sigmoid(_dot(cat, Wg[:]) + bg[:][None, :])
    fused = gate * h2s + (1.0 - gate) * h2f
    onehot = (batch_ref[:] ==
              lax.broadcasted_iota(jnp.int32, (N, B), 1)).astype(jnp.float32)
    sums = lax.dot_general(onehot, fused, (((0,), (0,)), ((), ())),
                           preferred_element_type=jnp.float32, precision=_HI)
    cnts = jnp.sum(onehot, axis=0)
    gp = sums / jnp.maximum(cnts, 1.0)[:, None]
    z = _dot(gp, Wp1[:]) + bp1[:][None, :]
    z = _bn(z, gp1[:][None, :], bp1bn[:][None, :])
    z = jnp.maximum(z, 0.0)
    z = _dot(z, Wp2[:]) + bp2[:][None, :]
    z = _bn(z, gp2[:][None, :], bp2bn[:][None, :])
    z = jnp.maximum(z, 0.0)
    out_ref[:] = _dot(z, Wp3[:]) + bp3[:][None, :]


_tc_head = pl.pallas_call(
    _tc_head_body,
    out_shape=jax.ShapeDtypeStruct((B, OUT), jnp.float32),
)


def kernel(x, edge_index_sc, edge_weight_sc, edge_index_fc, edge_weight_fc,
           batch, params):
    p = params
    i32 = jnp.int32
    src_sc = edge_index_sc[0].astype(i32)
    dst_sc = edge_index_sc[1].astype(i32)
    src_fc = edge_index_fc[0].astype(i32)
    dst_fc = edge_index_fc[1].astype(i32)

    npad = EP - E
    pad_idx = (jnp.arange(npad, dtype=i32) * 61) % N
    zpad = jnp.zeros((npad,), jnp.float32)
    src_all = jnp.concatenate(
        [src_sc, pad_idx, src_fc + N, pad_idx + N])
    dst_all = jnp.concatenate([dst_sc, pad_idx, dst_fc, pad_idx])
    w_all = jnp.concatenate(
        [edge_weight_sc.astype(jnp.float32), zpad,
         edge_weight_fc.astype(jnp.float32), zpad])

    src_r = src_all.reshape(2, NT, EPT_CH, C)
    dst_r = dst_all.reshape(2, NT, EPT_CH, C)
    w_r = w_all.reshape(2, NT, EPT_CH, C)

    deg2 = _sc_deg(dst_r, w_r)

    h, dinv2 = _tc_enc(x, deg2, p['We1'], p['be1'], p['ge1'], p['bbe1'],
                       p['We2'], p['be2'])
    dinv = [dinv2[0], dinv2[1]]

    hw0 = [_tc_prep(h, dinv[0], p['Wsc0']), _tc_prep(h, dinv[1], p['Wfc0'])]
    acc0 = _sc_mp(jnp.concatenate(hw0, axis=0), src_r, dst_r, w_r)

    hw1 = [_tc_mid(acc0[0], hw0[0], dinv[0], p['bsc0'], p['g_sc0'],
                   p['b_sc0'], p['Wsc1']),
           _tc_mid(acc0[1], hw0[1], dinv[1], p['bfc0'], p['g_fc0'],
                   p['b_fc0'], p['Wfc1'])]
    acc1 = _sc_mp(jnp.concatenate(hw1, axis=0), src_r, dst_r, w_r)

    h2s = _tc_lay2(acc1[0], hw1[0], dinv[0], p['bsc1'], p['g_sc1'], p['b_sc1'])
    h2f = _tc_lay2(acc1[1], hw1[1], dinv[1], p['bfc1'], p['g_fc1'], p['b_fc1'])

    return _tc_head(h2s, h2f, batch.astype(i32).reshape(N, 1),
                    p['Wg'], p['bg'],
                    p['Wp1'], p['bp1'], p['gp1'], p['bp1_bn'],
                    p['Wp2'], p['bp2'], p['gp2'], p['bp2_bn'],
                    p['Wp3'], p['bp3'])
